# baseline (speedup 1.0000x reference)
import jax, jax.numpy as jnp
from jax.experimental import pallas as pl


def _noop_body(x_ref, o_ref):
    o_ref[...] = x_ref[...]


def kernel(H_K_prev, edgelists, W_l, W_r, b_sage, W_proj, b_proj, W_ih, W_hh, b_ih, b_hh):
    K, N, d = H_K_prev.shape
    steps = []
    for k in range(K):
        x = H_K_prev[k]
        src = edgelists[k, 0]
        dst = edgelists[k, 1]
        msgs = jnp.take(x, src, axis=0)
        summed = jax.ops.segment_sum(msgs, dst, num_segments=N)
        deg = jax.ops.segment_sum(jnp.ones(src.shape[0], dtype=x.dtype), dst, num_segments=N)
        mean = summed / jnp.clip(deg, 1.0)[:, None]
        agg = mean @ W_l.T + x @ W_r.T + b_sage
        h_k = jnp.concatenate([x, agg], axis=1) @ W_proj.T + b_proj
        steps.append(h_k)
    seq = jnp.stack(steps, axis=1)
    h = jnp.zeros((N, d), dtype=seq.dtype)
    c = jnp.zeros((N, d), dtype=seq.dtype)
    for t in range(K):
        gates = seq[:, t] @ W_ih.T + b_ih + h @ W_hh.T + b_hh
        i, f, g, o = jnp.split(gates, 4, axis=1)
        i = jax.nn.sigmoid(i)
        f = jax.nn.sigmoid(f)
        g = jnp.tanh(g)
        o = jax.nn.sigmoid(o)
        c = f * c + i * g
        h = o * jnp.tanh(c)
    out = jax.nn.relu(h)[None]
    return pl.pallas_call(_noop_body, out_shape=jax.ShapeDtypeStruct(out.shape, out.dtype))(out)


# speedup vs baseline: 1.0063x; 1.0063x over previous
import functools

import jax
import jax.numpy as jnp
from jax import lax
from jax.experimental import pallas as pl
from jax.experimental.pallas import tpu as pltpu
from jax.experimental.pallas import tpu_sc as plsc

K = 4
N = 10000
D = 256
E = 160000

NC = 2
NS = 16
L = 16
NW = NC * NS

NPAD = 10240
OWN = NPAD // NW
NVEC = E // L
VEC_A = 313
VEC_B = 312
EPW_A = VEC_A * L
EPW_B = VEC_B * L
SEGV = 80
NSEG = 4
CAP = SEGV * L
NBLK = NW * NSEG
CHK = 16
SMASK = (1 << 14) - 1


def _sc_bin(src_flat, dst_flat):
    mesh = plsc.VectorSubcoreMesh(core_axis_name="c", subcore_axis_name="s")

    @functools.partial(
        pl.kernel,
        mesh=mesh,
        out_type=[
            jax.ShapeDtypeStruct((K, NBLK, NW, CAP), jnp.int32),
            jax.ShapeDtypeStruct((K, NBLK * NW), jnp.int32),
        ],
        scratch_types=[
            pltpu.VMEM((EPW_A,), jnp.int32),
            pltpu.VMEM((EPW_A,), jnp.int32),
            pltpu.VMEM((CAP,), jnp.int32),
            pltpu.VMEM((CAP,), jnp.int32),
            pltpu.VMEM((NW, CAP), jnp.int32),
            pltpu.VMEM((32,), jnp.int32),
            pltpu.SMEM((NW,), jnp.int32),
            pltpu.SemaphoreType.DMA,
        ],
    )
    def bink(srcf_hbm, dstf_hbm, rec_hbm, cnt_hbm,
             src_v, dst_v, pk_v, ow_v, reg_v, cv_v, cnt_m, sem):
        c = lax.axis_index("c")
        s = lax.axis_index("s")
        w = s * NC + c
        lanev = lax.broadcasted_iota(jnp.int32, (L,), 0)
        is_a = w < 16
        nvec = jnp.where(is_a, VEC_A, VEC_B)
        start = jnp.where(is_a, w * EPW_A,
                          16 * EPW_A + (w - 16) * EPW_B)

        for k in range(K):
            off = k * E + start

            @pl.when(is_a)
            def _():
                pltpu.sync_copy(srcf_hbm.at[pl.ds(off, EPW_A)], src_v)
                pltpu.sync_copy(dstf_hbm.at[pl.ds(off, EPW_A)], dst_v)

            @pl.when(jnp.logical_not(is_a))
            def _():
                pltpu.sync_copy(srcf_hbm.at[pl.ds(off, EPW_B)],
                                src_v.at[pl.ds(0, EPW_B)])
                pltpu.sync_copy(dstf_hbm.at[pl.ds(off, EPW_B)],
                                dst_v.at[pl.ds(0, EPW_B)])

            for seg in range(NSEG):
                segoff = seg * SEGV
                n_g = jnp.minimum(SEGV, nvec - segoff)

                def rst(o, _):
                    cnt_m[o] = 0
                    return 0
                lax.fori_loop(0, NW, rst, 0)

                def pre(i, _):
                    dv = dst_v[pl.ds((segoff + i) * L, L)]
                    sv = src_v[pl.ds((segoff + i) * L, L)]
                    ov = lax.shift_right_logical(dv * 6554, 21)
                    jloc = dv - ov * OWN
                    pk_v[pl.ds(i * L, L)] = (jloc << 14) | sv
                    ow_v[pl.ds(i * L, L)] = ov
                    return 0
                lax.fori_loop(0, n_g, pre, 0)

                def route(i, _):
                    ovec = ow_v[pl.ds(i * L, L)]
                    pvec = pk_v[pl.ds(i * L, L)]
                    for l in range(L):
                        o = ovec[l]
                        cnt = cnt_m[o]
                        fill = cnt & (L - 1)
                        gbase = pl.multiple_of(cnt - fill, L)
                        gsl = pl.ds(gbase, L)
                        cur = reg_v[o, gsl]
                        reg_v[o, gsl] = jnp.where(lanev == fill,
                                                  jnp.full((L,), pvec[l],
                                                           jnp.int32), cur)
                        cnt_m[o] = cnt + 1
                    return 0
                lax.fori_loop(0, n_g, route, 0)

                b = w * NSEG + seg
                pltpu.sync_copy(reg_v, rec_hbm.at[k, b])

                def mkc(q, _):
                    v = jnp.zeros((L,), jnp.int32)
                    for l in range(L):
                        v = jnp.where(lanev == l, cnt_m[q * L + l], v)
                    cv_v[pl.ds(q * L, L)] = v
                    return 0
                lax.fori_loop(0, NW // L, mkc, 0)
                pltpu.sync_copy(cv_v, cnt_hbm.at[k, pl.ds(b * NW, NW)])

    return bink(src_flat, dst_flat)


def _sc_accumulate(h, records, counts_t):
    mesh = plsc.VectorSubcoreMesh(core_axis_name="c", subcore_axis_name="s")

    @functools.partial(
        pl.kernel,
        mesh=mesh,
        out_type=[
            jax.ShapeDtypeStruct((K, NPAD, D), jnp.float32),
            jax.ShapeDtypeStruct((K, NPAD, L), jnp.float32),
        ],
        scratch_types=[
            pltpu.VMEM((OWN, D), jnp.float32),
            pltpu.VMEM((OWN, L), jnp.float32),
            pltpu.VMEM((CAP,), jnp.int32),
            pltpu.VMEM((CAP,), jnp.int32),
            pltpu.VMEM((NBLK + 16,), jnp.int32),
            pltpu.VMEM((CHK, D), jnp.float32),
            pltpu.SemaphoreType.DMA,
        ],
    )
    def acck(h_hbm, rec_hbm, cnt_hbm, sum_hbm, deg_hbm,
             acc_v, dacc_v, rb_v, jl_v, cn_v, rows_v, sem):
        c = lax.axis_index("c")
        s = lax.axis_index("s")
        o = s * NC + c
        base = o * OWN
        lanev = lax.broadcasted_iota(jnp.int32, (L,), 0)
        e0 = jnp.where(lanev == 0, 1.0, 0.0).astype(jnp.float32)

        for k in range(K):
            def zacc(i, _):
                for u in range(D // L):
                    acc_v[i, pl.ds(u * L, L)] = jnp.zeros((L,), jnp.float32)
                dacc_v[i, :] = jnp.zeros((L,), jnp.float32)
                return 0
            lax.fori_loop(0, OWN, zacc, 0)

            pltpu.sync_copy(cnt_hbm.at[k, pl.ds(o * NBLK, NBLK)],
                            cn_v.at[pl.ds(0, NBLK)])
            h_k = h_hbm.at[k]

            def block(b, _):
                pltpu.sync_copy(rec_hbm.at[k, b, o], rb_v)
                cvec = cn_v[pl.ds(b, L)]
                cnt = cvec[0]

                def unpack(i, _):
                    pv = rb_v[pl.ds(i * L, L)]
                    jl_v[pl.ds(i * L, L)] = lax.shift_right_logical(pv, 14)
                    rb_v[pl.ds(i * L, L)] = jnp.minimum(pv & SMASK, N - 1)
                    return 0
                lax.fori_loop(0, CAP // L, unpack, 0)

                nchunk = (cnt + CHK - 1) // CHK

                def chunk(t, _):
                    pltpu.async_copy(
                        h_k.at[rb_v.at[pl.ds(t * CHK, CHK)]],
                        rows_v, sem).wait()
                    jv = jl_v[pl.ds(t * CHK, L)]
                    eb = t * CHK
                    for l in range(L):
                        @pl.when(eb + l < cnt)
                        def _():
                            jr = jv[l]
                            for u in range(D // L):
                                sl = pl.ds(u * L, L)
                                acc_v[jr, sl] = (acc_v[jr, sl]
                                                 + rows_v[l, sl])
                            dacc_v[jr, :] = dacc_v[jr, :] + e0
                    return 0
                lax.fori_loop(0, nchunk, chunk, 0)
                return 0
            lax.fori_loop(0, NBLK, block, 0)

            pltpu.sync_copy(acc_v.at[pl.ds(0, OWN)],
                            sum_hbm.at[k, pl.ds(base, OWN)])
            pltpu.sync_copy(dacc_v.at[pl.ds(0, OWN)],
                            deg_hbm.at[k, pl.ds(base, OWN)])

    return acck(h, records, counts_t)


R = 256


def _tc_body(x_ref, sum_ref, deg_ref, wl_ref, wr_ref, wp_ref, wih_ref,
             whh_ref, bs_ref, bp_ref, bg_ref, out_ref):
    f32 = jnp.float32
    wl = wl_ref[...]
    wr = wr_ref[...]
    wp = wp_ref[...]
    wih = wih_ref[...]
    whh = whh_ref[...]
    bs = bs_ref[...]
    bp = bp_ref[...]
    bg = bg_ref[...]
    dn = (((1,), (1,)), ((), ()))

    h = jnp.zeros((R, D), f32)
    cst = jnp.zeros((R, D), f32)
    for t in range(K):
        x = x_ref[t]
        degc = jnp.maximum(deg_ref[:, t:t + 1], 1.0)
        mean = sum_ref[t] / degc
        agg = (lax.dot_general(mean, wl, dn, preferred_element_type=f32)
               + lax.dot_general(x, wr, dn, preferred_element_type=f32) + bs)
        seq_t = (lax.dot_general(x, wp[:, :D], dn, preferred_element_type=f32)
                 + lax.dot_general(agg, wp[:, D:], dn,
                                   preferred_element_type=f32) + bp)
        gates = (lax.dot_general(seq_t, wih, dn, preferred_element_type=f32)
                 + lax.dot_general(h, whh, dn, preferred_element_type=f32)
                 + bg)
        i = jax.nn.sigmoid(gates[:, :D])
        f = jax.nn.sigmoid(gates[:, D:2 * D])
        g = jnp.tanh(gates[:, 2 * D:3 * D])
        o = jax.nn.sigmoid(gates[:, 3 * D:])
        cst = f * cst + i * g
        h = o * jnp.tanh(cst)
    out_ref[0] = jnp.maximum(h, 0.0)


def _tc_dense(h, summed, deg_t, W_l, W_r, W_proj, W_ih, W_hh, bs, bp, bg):
    grid = (NPAD // R,)
    return pl.pallas_call(
        _tc_body,
        grid=grid,
        in_specs=[
            pl.BlockSpec((K, R, D), lambda i: (0, i, 0)),
            pl.BlockSpec((K, R, D), lambda i: (0, i, 0)),
            pl.BlockSpec((R, K), lambda i: (i, 0)),
            pl.BlockSpec((D, D), lambda i: (0, 0)),
            pl.BlockSpec((D, D), lambda i: (0, 0)),
            pl.BlockSpec((D, 2 * D), lambda i: (0, 0)),
            pl.BlockSpec((4 * D, D), lambda i: (0, 0)),
            pl.BlockSpec((4 * D, D), lambda i: (0, 0)),
            pl.BlockSpec((1, D), lambda i: (0, 0)),
            pl.BlockSpec((1, D), lambda i: (0, 0)),
            pl.BlockSpec((1, 4 * D), lambda i: (0, 0)),
        ],
        out_specs=pl.BlockSpec((1, R, D), lambda i: (0, i, 0)),
        out_shape=jax.ShapeDtypeStruct((1, N, D), jnp.float32),
    )(h, summed, deg_t, W_l, W_r, W_proj, W_ih, W_hh, bs, bp, bg)


def kernel(H_K_prev, edgelists, W_l, W_r, b_sage, W_proj, b_proj,
           W_ih, W_hh, b_ih, b_hh):
    src_flat = edgelists[:, 0].reshape(K * E)
    dst_flat = edgelists[:, 1].reshape(K * E)
    records, counts = _sc_bin(src_flat, dst_flat)
    counts_t = counts.reshape(K, NBLK, NW).transpose(0, 2, 1).reshape(
        K, NW * NBLK)
    summed, deg = _sc_accumulate(H_K_prev, records, counts_t)
    deg_t = deg[:, :, 0].T
    bs = b_sage.reshape(1, D)
    bp = b_proj.reshape(1, D)
    bg = (b_ih + b_hh).reshape(1, 4 * D)
    return _tc_dense(H_K_prev, summed, deg_t, W_l, W_r, W_proj,
                     W_ih, W_hh, bs, bp, bg)


# speedup vs baseline: 1.2490x; 1.2412x over previous
import functools

import jax
import jax.numpy as jnp
from jax import lax
from jax.experimental import pallas as pl
from jax.experimental.pallas import tpu as pltpu
from jax.experimental.pallas import tpu_sc as plsc

K = 4
N = 10000
D = 256
E = 160000

NC = 2
NS = 16
L = 16
NW = NC * NS

NPAD = 10240
OWN = NPAD // NW
NVEC = E // L
VEC_A = 313
VEC_B = 312
EPW_A = VEC_A * L
EPW_B = VEC_B * L
SEGV = 80
NSEG = 4
CAP = SEGV * L
NBLK = NW * NSEG
CHK = 8
SMASK = (1 << 14) - 1


def _sc_bin(src_flat, dst_flat):
    mesh = plsc.VectorSubcoreMesh(core_axis_name="c", subcore_axis_name="s")

    @functools.partial(
        pl.kernel,
        mesh=mesh,
        out_type=[
            jax.ShapeDtypeStruct((K, NBLK, NW, CAP), jnp.int32),
            jax.ShapeDtypeStruct((K, NBLK * NW), jnp.int32),
        ],
        scratch_types=[
            pltpu.VMEM((EPW_A,), jnp.int32),
            pltpu.VMEM((EPW_A,), jnp.int32),
            pltpu.VMEM((CAP,), jnp.int32),
            pltpu.VMEM((CAP,), jnp.int32),
            pltpu.VMEM((NW, CAP), jnp.int32),
            pltpu.VMEM((32,), jnp.int32),
            pltpu.SMEM((NW,), jnp.int32),
            pltpu.SemaphoreType.DMA,
        ],
    )
    def bink(srcf_hbm, dstf_hbm, rec_hbm, cnt_hbm,
             src_v, dst_v, pk_v, ow_v, reg_v, cv_v, cnt_m, sem):
        c = lax.axis_index("c")
        s = lax.axis_index("s")
        w = s * NC + c
        lanev = lax.broadcasted_iota(jnp.int32, (L,), 0)
        is_a = w < 16
        nvec = jnp.where(is_a, VEC_A, VEC_B)
        start = jnp.where(is_a, w * EPW_A,
                          16 * EPW_A + (w - 16) * EPW_B)

        for k in range(K):
            off = k * E + start

            @pl.when(is_a)
            def _():
                pltpu.sync_copy(srcf_hbm.at[pl.ds(off, EPW_A)], src_v)
                pltpu.sync_copy(dstf_hbm.at[pl.ds(off, EPW_A)], dst_v)

            @pl.when(jnp.logical_not(is_a))
            def _():
                pltpu.sync_copy(srcf_hbm.at[pl.ds(off, EPW_B)],
                                src_v.at[pl.ds(0, EPW_B)])
                pltpu.sync_copy(dstf_hbm.at[pl.ds(off, EPW_B)],
                                dst_v.at[pl.ds(0, EPW_B)])

            for seg in range(NSEG):
                segoff = seg * SEGV
                n_g = jnp.minimum(SEGV, nvec - segoff)

                def rst(o, _):
                    cnt_m[o] = 0
                    return 0
                lax.fori_loop(0, NW, rst, 0)

                def pre(i, _):
                    dv = dst_v[pl.ds((segoff + i) * L, L)]
                    sv = src_v[pl.ds((segoff + i) * L, L)]
                    ov = lax.shift_right_logical(dv * 6554, 21)
                    jloc = dv - ov * OWN
                    pk_v[pl.ds(i * L, L)] = (jloc << 14) | sv
                    ow_v[pl.ds(i * L, L)] = ov
                    return 0
                lax.fori_loop(0, n_g, pre, 0)

                def route(i, _):
                    ovec = ow_v[pl.ds(i * L, L)]
                    pvec = pk_v[pl.ds(i * L, L)]
                    for l in range(L):
                        o = ovec[l]
                        cnt = cnt_m[o]
                        fill = cnt & (L - 1)
                        gbase = pl.multiple_of(cnt - fill, L)
                        gsl = pl.ds(gbase, L)
                        cur = reg_v[o, gsl]
                        reg_v[o, gsl] = jnp.where(lanev == fill,
                                                  jnp.full((L,), pvec[l],
                                                           jnp.int32), cur)
                        cnt_m[o] = cnt + 1
                    return 0
                lax.fori_loop(0, n_g, route, 0)

                b = w * NSEG + seg
                pltpu.sync_copy(reg_v, rec_hbm.at[k, b])

                def mkc(q, _):
                    v = jnp.zeros((L,), jnp.int32)
                    for l in range(L):
                        v = jnp.where(lanev == l, cnt_m[q * L + l], v)
                    cv_v[pl.ds(q * L, L)] = v
                    return 0
                lax.fori_loop(0, NW // L, mkc, 0)
                pltpu.sync_copy(cv_v, cnt_hbm.at[k, pl.ds(b * NW, NW)])

    return bink(src_flat, dst_flat)


def _sc_accumulate(h, records, counts_t):
    mesh = plsc.VectorSubcoreMesh(core_axis_name="c", subcore_axis_name="s")

    @functools.partial(
        pl.kernel,
        mesh=mesh,
        out_type=[
            jax.ShapeDtypeStruct((K, NPAD, D), jnp.float32),
            jax.ShapeDtypeStruct((K, NPAD, L), jnp.float32),
        ],
        scratch_types=[
            pltpu.VMEM((OWN, D), jnp.float32),
            pltpu.VMEM((OWN, L), jnp.float32),
            pltpu.VMEM((CAP + L,), jnp.int32),
            pltpu.VMEM((CAP + L,), jnp.int32),
            pltpu.VMEM((NBLK + 16,), jnp.int32),
            pltpu.VMEM((2, CHK, D), jnp.float32),
            pltpu.SemaphoreType.DMA,
            pltpu.SemaphoreType.DMA,
        ],
    )
    def acck(h_hbm, rec_hbm, cnt_hbm, sum_hbm, deg_hbm,
             acc_v, dacc_v, rb_v, jl_v, cn_v, rows_v, sem0, sem1):
        c = lax.axis_index("c")
        s = lax.axis_index("s")
        o = s * NC + c
        base = o * OWN
        lanev = lax.broadcasted_iota(jnp.int32, (L,), 0)
        e0 = jnp.where(lanev == 0, 1.0, 0.0).astype(jnp.float32)

        for k in range(K):
            def zacc(i, _):
                for u in range(D // L):
                    acc_v[i, pl.ds(u * L, L)] = jnp.zeros((L,), jnp.float32)
                dacc_v[i, :] = jnp.zeros((L,), jnp.float32)
                return 0
            lax.fori_loop(0, OWN, zacc, 0)

            pltpu.sync_copy(cnt_hbm.at[k, pl.ds(o * NBLK, NBLK)],
                            cn_v.at[pl.ds(0, NBLK)])
            h_k = h_hbm.at[k]

            def block(b, _):
                pltpu.sync_copy(rec_hbm.at[k, b, o], rb_v.at[pl.ds(0, CAP)])
                cvec = cn_v[pl.ds(b, L)]
                cnt = cvec[0]

                def unpack(i, _):
                    pv = rb_v[pl.ds(i * L, L)]
                    jl_v[pl.ds(i * L, L)] = lax.shift_right_logical(pv, 14)
                    rb_v[pl.ds(i * L, L)] = jnp.minimum(pv & SMASK, N - 1)
                    return 0
                lax.fori_loop(0, CAP // L, unpack, 0)

                nchunk = (cnt + CHK - 1) // CHK

                @pl.when(nchunk > 0)
                def _():
                    pltpu.async_copy(
                        h_k.at[rb_v.at[pl.ds(0, CHK)]],
                        rows_v.at[0], sem0)

                def chunk(t, _):
                    par = t & 1
                    nxt = (t + 1) * CHK

                    @pl.when((t + 1 < nchunk) & (par == 0))
                    def _():
                        pltpu.async_copy(
                            h_k.at[rb_v.at[pl.ds(nxt, CHK)]],
                            rows_v.at[1], sem1)

                    @pl.when((t + 1 < nchunk) & (par == 1))
                    def _():
                        pltpu.async_copy(
                            h_k.at[rb_v.at[pl.ds(nxt, CHK)]],
                            rows_v.at[0], sem0)

                    @pl.when(par == 0)
                    def _():
                        pltpu.make_async_copy(
                            h_k.at[rb_v.at[pl.ds(t * CHK, CHK)]],
                            rows_v.at[0], sem0).wait()

                    @pl.when(par == 1)
                    def _():
                        pltpu.make_async_copy(
                            h_k.at[rb_v.at[pl.ds(t * CHK, CHK)]],
                            rows_v.at[1], sem1).wait()

                    jv = jl_v[pl.ds(t * CHK, L)]
                    eb = t * CHK
                    for l in range(CHK):
                        @pl.when(eb + l < cnt)
                        def _():
                            jr = jv[l]
                            for u in range(D // L):
                                sl = pl.ds(u * L, L)
                                acc_v[jr, sl] = (acc_v[jr, sl]
                                                 + rows_v[par, l, sl])
                            dacc_v[jr, :] = dacc_v[jr, :] + e0
                    return 0
                lax.fori_loop(0, nchunk, chunk, 0)
                return 0
            lax.fori_loop(0, NBLK, block, 0)

            pltpu.sync_copy(acc_v.at[pl.ds(0, OWN)],
                            sum_hbm.at[k, pl.ds(base, OWN)])
            pltpu.sync_copy(dacc_v.at[pl.ds(0, OWN)],
                            deg_hbm.at[k, pl.ds(base, OWN)])

    return acck(h, records, counts_t)


R = 256


def _tc_body(x_ref, sum_ref, deg_ref, wl_ref, wr_ref, wp_ref, wih_ref,
             whh_ref, bs_ref, bp_ref, bg_ref, out_ref):
    f32 = jnp.float32
    bf = jnp.bfloat16
    wl = wl_ref[...].astype(bf)
    wr = wr_ref[...].astype(bf)
    wp = wp_ref[...].astype(bf)
    wih = wih_ref[...].astype(bf)
    whh = whh_ref[...].astype(bf)
    bs = bs_ref[...]
    bp = bp_ref[...]
    bg = bg_ref[...]
    dn = (((1,), (1,)), ((), ()))

    def mm(a, b):
        return lax.dot_general(a.astype(bf), b, dn,
                               preferred_element_type=f32)

    h = jnp.zeros((R, D), f32)
    cst = jnp.zeros((R, D), f32)
    for t in range(K):
        x = x_ref[t]
        degc = jnp.maximum(deg_ref[:, t:t + 1], 1.0)
        mean = sum_ref[t] / degc
        agg = mm(mean, wl) + mm(x, wr) + bs
        seq_t = mm(x, wp[:, :D]) + mm(agg, wp[:, D:]) + bp
        gates = mm(seq_t, wih) + mm(h, whh) + bg
        i = jax.nn.sigmoid(gates[:, :D])
        f = jax.nn.sigmoid(gates[:, D:2 * D])
        g = jnp.tanh(gates[:, 2 * D:3 * D])
        o = jax.nn.sigmoid(gates[:, 3 * D:])
        cst = f * cst + i * g
        h = o * jnp.tanh(cst)
    out_ref[0] = jnp.maximum(h, 0.0)


def _tc_dense(h, summed, deg_t, W_l, W_r, W_proj, W_ih, W_hh, bs, bp, bg):
    grid = (NPAD // R,)
    return pl.pallas_call(
        _tc_body,
        grid=grid,
        in_specs=[
            pl.BlockSpec((K, R, D), lambda i: (0, i, 0)),
            pl.BlockSpec((K, R, D), lambda i: (0, i, 0)),
            pl.BlockSpec((R, K), lambda i: (i, 0)),
            pl.BlockSpec((D, D), lambda i: (0, 0)),
            pl.BlockSpec((D, D), lambda i: (0, 0)),
            pl.BlockSpec((D, 2 * D), lambda i: (0, 0)),
            pl.BlockSpec((4 * D, D), lambda i: (0, 0)),
            pl.BlockSpec((4 * D, D), lambda i: (0, 0)),
            pl.BlockSpec((1, D), lambda i: (0, 0)),
            pl.BlockSpec((1, D), lambda i: (0, 0)),
            pl.BlockSpec((1, 4 * D), lambda i: (0, 0)),
        ],
        out_specs=pl.BlockSpec((1, R, D), lambda i: (0, i, 0)),
        out_shape=jax.ShapeDtypeStruct((1, N, D), jnp.float32),
    )(h, summed, deg_t, W_l, W_r, W_proj, W_ih, W_hh, bs, bp, bg)


def kernel(H_K_prev, edgelists, W_l, W_r, b_sage, W_proj, b_proj,
           W_ih, W_hh, b_ih, b_hh):
    src_flat = edgelists[:, 0].reshape(K * E)
    dst_flat = edgelists[:, 1].reshape(K * E)
    records, counts = _sc_bin(src_flat, dst_flat)
    counts_t = counts.reshape(K, NBLK, NW).transpose(0, 2, 1).reshape(
        K, NW * NBLK)
    summed, deg = _sc_accumulate(H_K_prev, records, counts_t)
    deg_t = deg[:, :, 0].T
    bs = b_sage.reshape(1, D)
    bp = b_proj.reshape(1, D)
    bg = (b_ih + b_hh).reshape(1, 4 * D)
    return _tc_dense(H_K_prev, summed, deg_t, W_l, W_r, W_proj,
                     W_ih, W_hh, bs, bp, bg)
